# Initial kernel scaffold; baseline (speedup 1.0000x reference)
import jax
import jax.numpy as jnp
from jax.experimental import pallas as pl


def kernel(er_input, seg_label, seg_logit, gt_boundary_seg):
    raise NotImplementedError("write your pallas kernel here")


# speedup vs baseline: 11.7836x; 11.7836x over previous
import jax
import jax.numpy as jnp
from jax.experimental import pallas as pl
from jax.experimental.pallas import tpu as pltpu

_T = 0.1
_EPS = 1e-8
_H = 64
_W = 64
_P = _H * _W
_C = 256
_OFFS = [(dh, dw) for dh in range(-2, 3) for dw in range(-2, 3)]


def _shift_flat(x, s):
    if s == 0:
        return x
    z = jnp.zeros(x.shape[:-1] + (abs(s),), x.dtype)
    if s > 0:
        return jnp.concatenate([x[..., s:], z], axis=-1)
    return jnp.concatenate([z, x[..., :s]], axis=-1)


def _loss_kernel(f_ref, lab_ref, logit_ref, gt_ref, out_ref):
    F = f_ref[...]
    lab = lab_ref[...]
    lg0 = logit_ref[0:1, :]
    lg1 = logit_ref[1:2, :]
    gt = gt_ref[...]

    col = jax.lax.broadcasted_iota(jnp.int32, (1, _P), 1) % _W
    wmask = {
        dw: jnp.logical_and(col + dw >= 0, col + dw < _W).astype(jnp.float32)
        for dw in range(-2, 3)
    }

    def box25(x):
        sh = x
        for dh in (-2, -1, 1, 2):
            sh = sh + _shift_flat(x, dh * _W)
        out = sh
        for dw in (-2, -1, 1, 2):
            out = out + _shift_flat(sh, dw) * wmask[dw]
        return out

    pred1 = lg1 > lg0
    edge = jnp.logical_and(gt != 0, gt != 255).astype(jnp.float32)
    c_cls = []
    for i in (0, 1):
        li = lab == i
        pi = pred1 if i == 1 else jnp.logical_not(pred1)
        c_cls.append(jnp.logical_and(li, pi).astype(jnp.float32))

    N = jnp.sqrt(jnp.sum(F * F, axis=0, keepdims=True))

    Dk, Nk = [], []
    for dh, dw in _OFFS:
        m = wmask[dw]
        Fs = _shift_flat(F, dh * _W + dw) * m
        Dk.append(jnp.sum(F * Fs, axis=0, keepdims=True))
        Nk.append(_shift_flat(N, dh * _W + dw) * m)

    total = jnp.float32(0.0)
    for i in (0, 1):
        ci = c_cls[i]
        cp = c_cls[1 - i]
        M = F * ci
        pvec = (box25(M) - M) * (1.0 / 25.0)
        fdotp = jnp.sum(F * pvec, axis=0, keepdims=True)
        pn = jnp.sqrt(jnp.sum(pvec * pvec, axis=0, keepdims=True))
        aden = ci * N + _EPS
        lpos = (ci * fdotp) / (aden * (pn + _EPS)) * (1.0 / _T)
        mx = lpos
        negs = []
        for k, (dh, dw) in enumerate(_OFFS):
            cpk = _shift_flat(cp, dh * _W + dw) * wmask[dw]
            nl = (ci * 2.0 * Dk[k] * cpk) / (
                aden * (2.0 * Nk[k] * cpk + _EPS)) * (1.0 / _T)
            negs.append(nl)
            mx = jnp.maximum(mx, nl)
        ssum = jnp.exp(lpos - mx)
        for nl in negs:
            ssum = ssum + jnp.exp(nl - mx)
        loss = mx + jnp.log(ssum) - lpos

        lmask = (lab == i).astype(jnp.float32)
        cnt = box25(lmask) - lmask
        pm = (cnt >= 1.0).astype(jnp.float32) * edge * lmask
        total = total + jnp.sum(loss * pm) / jnp.maximum(jnp.sum(pm), 1.0)

    out_ref[...] = jnp.broadcast_to(total, (1, 1))


def kernel(er_input, seg_label, seg_logit, gt_boundary_seg):
    F = er_input.reshape(_C, _P)
    lab = seg_label.reshape(1, _P).astype(jnp.int32)
    logit = seg_logit.reshape(2, _P)
    gt = gt_boundary_seg.reshape(1, _P).astype(jnp.int32)
    out = pl.pallas_call(
        _loss_kernel,
        out_shape=jax.ShapeDtypeStruct((1, 1), jnp.float32),
    )(F, lab, logit, gt)
    return out.reshape(())
